# initial kernel scaffold (unmeasured)
import jax
import jax.numpy as jnp
from jax import lax
from jax.experimental import pallas as pl
from jax.experimental.pallas import tpu as pltpu

N_DEV = 32


def kernel(x, w_mat, scale_x, scale_w):
    m_per, k = x.shape
    n = w_mat.shape[1]
    n_per = n // N_DEV

    def body(x_ref, w_ref, sx_ref, sw_ref, out_ref, y_ref, send_sems, recv_sems):
        my = lax.axis_index("i")

        acc = jnp.dot(
            x_ref[...], w_ref[...],
            preferred_element_type=jnp.float32,
            precision=lax.Precision.DEFAULT,
        )
        scale = sx_ref[0] * sw_ref[0]
        y_ref[...] = jnp.maximum(acc * scale, 0.0)

        out_ref[pl.ds(my * m_per, m_per), :] = y_ref[:, pl.ds(my * n_per, n_per)]

        rdmas = []
        for d in range(1, N_DEV):
            dst = lax.rem(my + d, N_DEV)
            rdma = pltpu.make_async_remote_copy(
                src_ref=y_ref.at[:, pl.ds(dst * n_per, n_per)],
                dst_ref=out_ref.at[pl.ds(my * m_per, m_per), :],
                send_sem=send_sems.at[d],
                recv_sem=recv_sems.at[d],
                device_id=(dst,),
                device_id_type=pl.DeviceIdType.MESH,
            )
            rdma.start()
            rdmas.append(rdma)

        for d in range(1, N_DEV):
            src = lax.rem(my - d + N_DEV, N_DEV)
            recv = pltpu.make_async_remote_copy(
                src_ref=y_ref.at[:, pl.ds(my * n_per, n_per)],
                dst_ref=out_ref.at[pl.ds(src * m_per, m_per), :],
                send_sem=send_sems.at[d],
                recv_sem=recv_sems.at[d],
                device_id=(src,),
                device_id_type=pl.DeviceIdType.MESH,
            )
            recv.wait_recv()

        for rdma in rdmas:
            rdma.wait_send()

    return pl.pallas_call(
        body,
        out_shape=jax.ShapeDtypeStruct((N_DEV * m_per, n_per), jnp.float32),
        in_specs=[
            pl.BlockSpec(memory_space=pltpu.VMEM),
            pl.BlockSpec(memory_space=pltpu.VMEM),
            pl.BlockSpec(memory_space=pltpu.SMEM),
            pl.BlockSpec(memory_space=pltpu.SMEM),
        ],
        out_specs=pl.BlockSpec(memory_space=pltpu.VMEM),
        scratch_shapes=[
            pltpu.VMEM((m_per, n), jnp.float32),
            pltpu.SemaphoreType.DMA((N_DEV,)),
            pltpu.SemaphoreType.DMA((N_DEV,)),
        ],
    )(x, w_mat, scale_x, scale_w)


# baseline (device time: 54892 ns/iter reference)
import jax
import jax.numpy as jnp
from jax import lax
from jax.experimental import pallas as pl
from jax.experimental.pallas import tpu as pltpu

N_DEV = 32


def kernel(x, w_mat, scale_x, scale_w):
    m_per, k = x.shape
    n = w_mat.shape[1]
    n_per = n // N_DEV

    def body(x_ref, w_ref, sx_ref, sw_ref, out_ref, y_ref, send_sems, recv_sems):
        my = lax.axis_index("i")

        acc = jnp.dot(
            x_ref[...], w_ref[...],
            preferred_element_type=jnp.float32,
            precision=lax.Precision.DEFAULT,
        )
        scale = sx_ref[0] * sw_ref[0]
        y = jnp.maximum(acc * scale, 0.0)

        for j in range(N_DEV):
            y_ref[j] = y[:, j * n_per:(j + 1) * n_per]

        out_ref[pl.ds(my * m_per, m_per), :] = y_ref[my]

        rdmas = []
        for d in range(1, N_DEV):
            dst = lax.rem(my + d, N_DEV)
            rdma = pltpu.make_async_remote_copy(
                src_ref=y_ref.at[dst],
                dst_ref=out_ref.at[pl.ds(my * m_per, m_per), :],
                send_sem=send_sems.at[d],
                recv_sem=recv_sems.at[d],
                device_id=(dst,),
                device_id_type=pl.DeviceIdType.MESH,
            )
            rdma.start()
            rdmas.append(rdma)

        for d in range(1, N_DEV):
            src = lax.rem(my - d + N_DEV, N_DEV)
            recv = pltpu.make_async_remote_copy(
                src_ref=y_ref.at[0],
                dst_ref=out_ref.at[pl.ds(src * m_per, m_per), :],
                send_sem=send_sems.at[d],
                recv_sem=recv_sems.at[d],
                device_id=(src,),
                device_id_type=pl.DeviceIdType.MESH,
            )
            recv.wait_recv()

        for rdma in rdmas:
            rdma.wait_send()

    return pl.pallas_call(
        body,
        out_shape=jax.ShapeDtypeStruct((N_DEV * m_per, n_per), jnp.float32),
        in_specs=[
            pl.BlockSpec(memory_space=pltpu.VMEM),
            pl.BlockSpec(memory_space=pltpu.VMEM),
            pl.BlockSpec(memory_space=pltpu.SMEM),
            pl.BlockSpec(memory_space=pltpu.SMEM),
        ],
        out_specs=pl.BlockSpec(memory_space=pltpu.VMEM),
        scratch_shapes=[
            pltpu.VMEM((N_DEV, m_per, n_per), jnp.float32),
            pltpu.SemaphoreType.DMA((N_DEV,)),
            pltpu.SemaphoreType.DMA((N_DEV,)),
        ],
        compiler_params=pltpu.CompilerParams(
            vmem_limit_bytes=96 * 1024 * 1024,
        ),
    )(x, w_mat, scale_x, scale_w)


# device time: 42281 ns/iter; 1.2983x vs baseline; 1.2983x over previous
import jax
import jax.numpy as jnp
from jax import lax
from jax.experimental import pallas as pl
from jax.experimental.pallas import tpu as pltpu

N_DEV = 32
N_PAIR = N_DEV // 2


def kernel(x, w_mat, scale_x, scale_w):
    m_per, k = x.shape
    n = w_mat.shape[1]
    n_per = n // N_DEV

    my_outer = lax.axis_index("i")
    pair_perm = lax.rem(
        my_outer // 2 + jnp.arange(N_PAIR, dtype=jnp.int32), N_PAIR
    )

    def body(perm_ref, x_ref, w_ref, sx_ref, sw_ref, out_ref,
             y_ref, send_sems, recv_sems):
        t = pl.program_id(0)
        my = lax.axis_index("i")
        j = perm_ref[t]

        @pl.when(t == 0)
        def _():
            barrier_sem = pltpu.get_barrier_semaphore()
            for d in range(1, N_DEV):
                pl.semaphore_signal(
                    barrier_sem, inc=1,
                    device_id=(lax.rem(my + d, N_DEV),),
                    device_id_type=pl.DeviceIdType.MESH,
                )
            pl.semaphore_wait(barrier_sem, N_DEV - 1)

        acc = jnp.dot(
            x_ref[...], w_ref[...],
            preferred_element_type=jnp.float32,
            precision=lax.Precision.DEFAULT,
        )
        yb = jnp.maximum(acc * (sx_ref[0] * sw_ref[0]), 0.0)

        for h in range(2):
            dst = 2 * j + h
            half = yb[:, h * n_per:(h + 1) * n_per]

            @pl.when(dst == my)
            def _():
                out_ref[pl.ds(my * m_per, m_per), :] = half

            @pl.when(dst != my)
            def _():
                y_ref[pl.ds(dst, 1)] = half[None]
                rdma = pltpu.make_async_remote_copy(
                    src_ref=y_ref.at[dst],
                    dst_ref=out_ref.at[pl.ds(my * m_per, m_per), :],
                    send_sem=send_sems.at[dst],
                    recv_sem=recv_sems.at[my],
                    device_id=(dst,),
                    device_id_type=pl.DeviceIdType.MESH,
                )
                rdma.start()

        @pl.when(t == N_PAIR - 1)
        def _():
            for d in range(1, N_DEV):
                src = lax.rem(my - d + N_DEV, N_DEV)
                recv = pltpu.make_async_remote_copy(
                    src_ref=y_ref.at[0],
                    dst_ref=out_ref.at[pl.ds(src * m_per, m_per), :],
                    send_sem=send_sems.at[src],
                    recv_sem=recv_sems.at[src],
                    device_id=(src,),
                    device_id_type=pl.DeviceIdType.MESH,
                )
                recv.wait_recv()
            for d in range(1, N_DEV):
                dst = lax.rem(my + d, N_DEV)
                send = pltpu.make_async_remote_copy(
                    src_ref=y_ref.at[dst],
                    dst_ref=out_ref.at[pl.ds(0, m_per), :],
                    send_sem=send_sems.at[dst],
                    recv_sem=recv_sems.at[dst],
                    device_id=(dst,),
                    device_id_type=pl.DeviceIdType.MESH,
                )
                send.wait_send()

    grid_spec = pltpu.PrefetchScalarGridSpec(
        num_scalar_prefetch=1,
        grid=(N_PAIR,),
        in_specs=[
            pl.BlockSpec((m_per, k), lambda t, perm: (0, 0)),
            pl.BlockSpec((k, 2 * n_per), lambda t, perm: (0, perm[t])),
            pl.BlockSpec(memory_space=pltpu.SMEM),
            pl.BlockSpec(memory_space=pltpu.SMEM),
        ],
        out_specs=pl.BlockSpec((N_DEV * m_per, n_per), lambda t, perm: (0, 0)),
        scratch_shapes=[
            pltpu.VMEM((N_DEV, m_per, n_per), jnp.float32),
            pltpu.SemaphoreType.DMA((N_DEV,)),
            pltpu.SemaphoreType.DMA((N_DEV,)),
        ],
    )

    return pl.pallas_call(
        body,
        out_shape=jax.ShapeDtypeStruct((N_DEV * m_per, n_per), jnp.float32),
        grid_spec=grid_spec,
        compiler_params=pltpu.CompilerParams(
            vmem_limit_bytes=96 * 1024 * 1024,
            collective_id=0,
        ),
    )(pair_perm, x, w_mat, scale_x, scale_w)
